# baseline (speedup 1.0000x reference)
import jax
import jax.numpy as jnp
from jax.experimental import pallas as pl
from jax.experimental.pallas import tpu as pltpu


R = 512
K = 8


def kernel(x, pos_embed):
    B, N, D = x.shape
    chunks_per_b = N // R
    n_chunks = B * chunks_per_b

    def body(x_hbm, pos_hbm, o_hbm, pos_vmem, in_vmem, out_vmem,
             pos_sem, in_sems, out_sems):
        def in_copy(i):
            b, j = divmod(i, chunks_per_b)
            return pltpu.make_async_copy(
                x_hbm.at[b, pl.ds(j * R, R), :],
                in_vmem.at[i % K],
                in_sems.at[i % K],
            )

        def out_copy(i):
            b, j = divmod(i, chunks_per_b)
            return pltpu.make_async_copy(
                out_vmem.at[i % K],
                o_hbm.at[b, pl.ds(j * R, R), :],
                out_sems.at[i % K],
            )

        pos_copy = pltpu.make_async_copy(
            pos_hbm.at[pl.ds(0, N), :], pos_vmem, pos_sem
        )
        pos_copy.start()
        for i in range(K):
            in_copy(i).start()
        pos_copy.wait()

        for i in range(n_chunks):
            slot = i % K
            in_copy(i).wait()
            if i >= K:
                out_copy(i - K).wait()
            j = i % chunks_per_b
            out_vmem[slot] = in_vmem[slot] + pos_vmem[pl.ds(j * R, R), :]
            out_copy(i).start()
            if i + K < n_chunks:
                in_copy(i + K).start()

        for i in range(n_chunks - K, n_chunks):
            out_copy(i).wait()

    return pl.pallas_call(
        body,
        in_specs=[
            pl.BlockSpec(memory_space=pl.ANY),
            pl.BlockSpec(memory_space=pl.ANY),
        ],
        out_specs=pl.BlockSpec(memory_space=pl.ANY),
        out_shape=jax.ShapeDtypeStruct((B, N, D), x.dtype),
        scratch_shapes=[
            pltpu.VMEM((N, D), x.dtype),
            pltpu.VMEM((K, R, D), x.dtype),
            pltpu.VMEM((K, R, D), x.dtype),
            pltpu.SemaphoreType.DMA,
            pltpu.SemaphoreType.DMA((K,)),
            pltpu.SemaphoreType.DMA((K,)),
        ],
    )(x, pos_embed)


# speedup vs baseline: 1.0151x; 1.0151x over previous
import jax
import jax.numpy as jnp
from jax.experimental import pallas as pl
from jax.experimental.pallas import tpu as pltpu


R = 1024
K = 5


def kernel(x, pos_embed):
    B, N, D = x.shape
    chunks_per_b = N // R
    n_chunks = B * chunks_per_b

    def body(x_hbm, pos_hbm, o_hbm, pos_vmem, in_vmem, out_vmem,
             pos_sem, in_sems, out_sems):
        def in_copy(i):
            b, j = divmod(i, chunks_per_b)
            return pltpu.make_async_copy(
                x_hbm.at[b, pl.ds(j * R, R), :],
                in_vmem.at[i % K],
                in_sems.at[i % K],
            )

        def out_copy(i):
            b, j = divmod(i, chunks_per_b)
            return pltpu.make_async_copy(
                out_vmem.at[i % K],
                o_hbm.at[b, pl.ds(j * R, R), :],
                out_sems.at[i % K],
            )

        pos_copy = pltpu.make_async_copy(
            pos_hbm.at[pl.ds(0, N), :], pos_vmem, pos_sem
        )
        pos_copy.start()
        for i in range(K):
            in_copy(i).start()
        pos_copy.wait()

        for i in range(n_chunks):
            slot = i % K
            in_copy(i).wait()
            if i >= K:
                out_copy(i - K).wait()
            j = i % chunks_per_b
            out_vmem[slot] = in_vmem[slot] + pos_vmem[pl.ds(j * R, R), :]
            out_copy(i).start()
            if i + K < n_chunks:
                in_copy(i + K).start()

        for i in range(n_chunks - K, n_chunks):
            out_copy(i).wait()

    return pl.pallas_call(
        body,
        in_specs=[
            pl.BlockSpec(memory_space=pl.ANY),
            pl.BlockSpec(memory_space=pl.ANY),
        ],
        out_specs=pl.BlockSpec(memory_space=pl.ANY),
        out_shape=jax.ShapeDtypeStruct((B, N, D), x.dtype),
        scratch_shapes=[
            pltpu.VMEM((N, D), x.dtype),
            pltpu.VMEM((K, R, D), x.dtype),
            pltpu.VMEM((K, R, D), x.dtype),
            pltpu.SemaphoreType.DMA,
            pltpu.SemaphoreType.DMA((K,)),
            pltpu.SemaphoreType.DMA((K,)),
        ],
    )(x, pos_embed)
